# Initial kernel scaffold; baseline (speedup 1.0000x reference)
import jax
import jax.numpy as jnp
from jax.experimental import pallas as pl


def kernel(x, edge_index, edge_type, graph_ids, head_ids, tail_ids, rel_labels, W1, W2, Wself1, Wself2, rel_emb, fc_W, fc_b):
    raise NotImplementedError("write your pallas kernel here")


# speedup vs baseline: 3.5417x; 3.5417x over previous
import functools

import jax
import jax.numpy as jnp
from jax.experimental import pallas as pl
from jax.experimental.pallas import tpu as pltpu

N = 10000
E = 320000
D = 128
R = 8
B = 200
G = 50

_INTERPRET = False


def _proj_body(h_ref, w_ref, wself_ref, hall_ref, self_ref):
    hb = h_ref[...]
    for r in range(R):
        hall_ref[:, r, :] = jnp.dot(hb, w_ref[r], preferred_element_type=jnp.float32)
    self_ref[...] = jnp.dot(hb, wself_ref[...], preferred_element_type=jnp.float32)


def _proj(h, W, Wself):
    bn = 1000
    return pl.pallas_call(
        _proj_body,
        grid=(N // bn,),
        in_specs=[
            pl.BlockSpec((bn, D), lambda i: (i, 0)),
            pl.BlockSpec((R, D, D), lambda i: (0, 0, 0)),
            pl.BlockSpec((D, D), lambda i: (0, 0)),
        ],
        out_specs=[
            pl.BlockSpec((bn, R, D), lambda i: (i, 0, 0)),
            pl.BlockSpec((bn, D), lambda i: (i, 0)),
        ],
        out_shape=[
            jax.ShapeDtypeStruct((N, R, D), jnp.float32),
            jax.ShapeDtypeStruct((N, D), jnp.float32),
        ],
        interpret=_INTERPRET,
    )(h, W, Wself)


def _combine_body(agg_ref, self_ref, deginv_ref, out_ref):
    out_ref[...] = jnp.maximum(
        agg_ref[...] * deginv_ref[...] + self_ref[...], 0.0)


def _combine(agg, selfp, deginv):
    bn = 2000
    return pl.pallas_call(
        _combine_body,
        grid=(N // bn,),
        in_specs=[
            pl.BlockSpec((bn, D), lambda i: (i, 0)),
            pl.BlockSpec((bn, D), lambda i: (i, 0)),
            pl.BlockSpec((bn, 1), lambda i: (i, 0)),
        ],
        out_specs=pl.BlockSpec((bn, D), lambda i: (i, 0)),
        out_shape=jax.ShapeDtypeStruct((N, D), jnp.float32),
        interpret=_INTERPRET,
    )(agg, selfp, deginv)


def _head_body(h1_ref, h2_ref, wa_ref, wb_ref, rel_ref, relemb_ref, wr_ref,
               fcb_ref, out_ref):
    i = pl.program_id(0)
    bn = h1_ref.shape[0]
    s = (jnp.dot(h1_ref[...], wa_ref[...], preferred_element_type=jnp.float32)
         + jnp.dot(h2_ref[...], wb_ref[...], preferred_element_type=jnp.float32))
    node = jax.lax.broadcasted_iota(jnp.int32, (bn, 1), 0) + i * bn
    ishead = (node % G == 0).astype(jnp.float32)
    istail = (node % G == 1).astype(jnp.float32)
    sel = jnp.concatenate(
        [jnp.full((bn, 1), 1.0 / G, jnp.float32), ishead, istail], axis=1)
    u = jnp.sum(s * sel, axis=1, keepdims=True)
    gid = (node // G)[:, 0]
    pool = (jax.lax.broadcasted_iota(jnp.int32, (B, bn), 0)
            == gid[None, :]).astype(jnp.float32)
    contrib = jnp.dot(pool, u, preferred_element_type=jnp.float32)

    @pl.when(i == 0)
    def _():
        relv = jnp.dot(relemb_ref[...], wr_ref[...],
                       preferred_element_type=jnp.float32)
        onehot = (jax.lax.broadcasted_iota(jnp.int32, (B, R), 1)
                  == rel_ref[...]).astype(jnp.float32)
        out_ref[...] = (jnp.dot(onehot, relv, preferred_element_type=jnp.float32)
                        + fcb_ref[0, 0])

    out_ref[...] += contrib


def _head(h1, h2, wa, wb, rel_labels, rel_emb, wr, fcb):
    bn = 2000
    return pl.pallas_call(
        _head_body,
        grid=(N // bn,),
        in_specs=[
            pl.BlockSpec((bn, D), lambda i: (i, 0)),
            pl.BlockSpec((bn, D), lambda i: (i, 0)),
            pl.BlockSpec((D, 3), lambda i: (0, 0)),
            pl.BlockSpec((D, 3), lambda i: (0, 0)),
            pl.BlockSpec((B, 1), lambda i: (0, 0)),
            pl.BlockSpec((R, 32), lambda i: (0, 0)),
            pl.BlockSpec((32, 1), lambda i: (0, 0)),
            pl.BlockSpec((1, 1), lambda i: (0, 0)),
        ],
        out_specs=pl.BlockSpec((B, 1), lambda i: (0, 0)),
        out_shape=jax.ShapeDtypeStruct((B, 1), jnp.float32),
        interpret=_INTERPRET,
    )(h1, h2, wa, wb, rel_labels, rel_emb, wr, fcb)


def kernel(x, edge_index, edge_type, graph_ids, head_ids, tail_ids, rel_labels,
           W1, W2, Wself1, Wself2, rel_emb, fc_W, fc_b):
    src = edge_index[0]
    dst = edge_index[1]

    deg = jax.ops.segment_sum(jnp.ones((E,), jnp.float32), dst, num_segments=N)
    deginv = (1.0 / jnp.maximum(deg, 1.0))[:, None]

    gidx = src * R + edge_type

    def layer(h, W, Wself):
        hall, selfp = _proj(h, W, Wself)
        msgs = hall.reshape(N * R, D)[gidx]
        agg = jax.ops.segment_sum(msgs, dst, num_segments=N)
        return _combine(agg, selfp, deginv)

    h1 = layer(x, W1, Wself1)
    h2 = layer(h1, W2, Wself2)

    wa = jnp.stack([fc_W[0:D, 0], fc_W[2 * D:3 * D, 0],
                    fc_W[4 * D:5 * D, 0]], axis=1)
    wb = jnp.stack([fc_W[D:2 * D, 0], fc_W[3 * D:4 * D, 0],
                    fc_W[5 * D:6 * D, 0]], axis=1)
    wr = fc_W[6 * D:6 * D + 32]
    return _head(h1, h2, wa, wb, rel_labels[:, None], rel_emb, wr,
                 fc_b.reshape(1, 1))


# speedup vs baseline: 9.9057x; 2.7969x over previous
import functools

import jax
import jax.numpy as jnp
from jax.experimental import pallas as pl
from jax.experimental.pallas import tpu as pltpu
from jax.experimental.pallas import tpu_sc as plsc

N = 10000
E = 320000
D = 128
R = 8
B = 200
G = 50

_INTERPRET = False

_NW = 32
_CH = 128
_NCH = 80
_EPT = _NCH * _CH
_EPAD = _EPT * _NW
_ROWS = 10112
_RPS = _ROWS // 16


def _sc_body(hall, src2, et2, dst2, acc_out,
             stage_a, stage_b, gidxv, dstv, rowsv, zbuf, acc_sh):
    c = jax.lax.axis_index("c")
    s = jax.lax.axis_index("s")
    wid = c * 16 + s
    ebase = wid * _NCH

    pltpu.sync_copy(dst2.at[pl.ds(ebase, _NCH)], dstv)

    zeros16 = jnp.zeros((16,), jnp.float32)

    def zero_zbuf(i, carry):
        zbuf[i // 8, pl.ds((i % 8) * 16, 16)] = zeros16
        return carry
    jax.lax.fori_loop(0, 16 * 8, zero_zbuf, 0)

    base = s * _RPS
    def zero_acc(i, carry):
        pltpu.sync_copy(zbuf, acc_sh.at[pl.ds(base + i * 16, 16)])
        return carry
    jax.lax.fori_loop(0, 39, zero_acc, 0)
    pltpu.sync_copy(zbuf.at[pl.ds(0, 8)], acc_sh.at[pl.ds(base + 624, 8)])

    def ggrp(g, carry):
        pltpu.sync_copy(src2.at[pl.ds(ebase + g * 8, 8)], stage_a)
        pltpu.sync_copy(et2.at[pl.ds(ebase + g * 8, 8)], stage_b)

        def gx(f, carry2):
            jj = f // 8
            sl = pl.ds((f % 8) * 16, 16)
            gidxv[g * 8 + jj, sl] = stage_a[jj, sl] * R + stage_b[jj, sl]
            return carry2
        jax.lax.fori_loop(0, 64, gx, 0)
        return carry
    jax.lax.fori_loop(0, _NCH // 8, ggrp, 0)

    plsc.subcore_barrier()

    def chunk(j, carry):
        pltpu.sync_copy(hall.at[gidxv.at[j]], rowsv)
        pltpu.sync_copy(rowsv, acc_sh.at[dstv.at[j]], add=True)
        return carry
    jax.lax.fori_loop(0, _NCH, chunk, 0)

    plsc.subcore_barrier()

    pltpu.sync_copy(acc_sh.at[pl.ds(base, _RPS)],
                    acc_out.at[c, pl.ds(base, _RPS)])


def _sc_pass(hall_flat, src2, et2, dst2):
    mesh = plsc.VectorSubcoreMesh(core_axis_name="c", subcore_axis_name="s")
    f = pl.kernel(
        _sc_body,
        out_type=jax.ShapeDtypeStruct((2, _ROWS, D), jnp.float32),
        mesh=mesh,
        scratch_types=[
            pltpu.VMEM((8, _CH), jnp.int32),
            pltpu.VMEM((8, _CH), jnp.int32),
            pltpu.VMEM((_NCH, _CH), jnp.int32),
            pltpu.VMEM((_NCH, _CH), jnp.int32),
            pltpu.VMEM((_CH, D), jnp.float32),
            pltpu.VMEM((16, D), jnp.float32),
            pltpu.VMEM_SHARED((_ROWS, D), jnp.float32),
        ],
        interpret=_INTERPRET,
    )
    return f(hall_flat, src2, et2, dst2)


def _deg_body(dst2, deg_out, dstv, onesbuf, zbuf, deg_sh):
    c = jax.lax.axis_index("c")
    s = jax.lax.axis_index("s")
    wid = c * 16 + s

    pltpu.sync_copy(dst2.at[pl.ds(wid * _NCH, _NCH)], dstv)

    zeros16 = jnp.zeros((16,), jnp.float32)
    ones16 = jnp.ones((16,), jnp.float32)

    def fillz(i, carry):
        zbuf[i // 8, pl.ds((i % 8) * 16, 16)] = zeros16
        return carry
    jax.lax.fori_loop(0, 16 * 8, fillz, 0)

    def fillo(i, carry):
        onesbuf[i // 8, pl.ds((i % 8) * 16, 16)] = ones16
        return carry
    jax.lax.fori_loop(0, _CH * 8, fillo, 0)

    base = s * _RPS
    def zero_deg(i, carry):
        pltpu.sync_copy(zbuf, deg_sh.at[pl.ds(base + i * 16, 16)])
        return carry
    jax.lax.fori_loop(0, 39, zero_deg, 0)
    pltpu.sync_copy(zbuf.at[pl.ds(0, 8)], deg_sh.at[pl.ds(base + 624, 8)])

    plsc.subcore_barrier()

    def chunk(j, carry):
        pltpu.sync_copy(onesbuf, deg_sh.at[dstv.at[j]], add=True)
        return carry
    jax.lax.fori_loop(0, _NCH, chunk, 0)

    plsc.subcore_barrier()

    pltpu.sync_copy(deg_sh.at[pl.ds(base, _RPS)],
                    deg_out.at[c, pl.ds(base, _RPS)])


def _deg_pass(dst2):
    mesh = plsc.VectorSubcoreMesh(core_axis_name="c", subcore_axis_name="s")
    f = pl.kernel(
        _deg_body,
        out_type=jax.ShapeDtypeStruct((2, _ROWS, D), jnp.float32),
        mesh=mesh,
        scratch_types=[
            pltpu.VMEM((_NCH, _CH), jnp.int32),
            pltpu.VMEM((_CH, D), jnp.float32),
            pltpu.VMEM((16, D), jnp.float32),
            pltpu.VMEM_SHARED((_ROWS, D), jnp.float32),
        ],
        interpret=_INTERPRET,
    )
    return f(dst2)


def _proj_body(h_ref, w_ref, wself_ref, hall_ref, self_ref):
    hb = h_ref[...]
    for r in range(R):
        hall_ref[:, r, :] = jnp.dot(hb, w_ref[r],
                                    preferred_element_type=jnp.float32)
    self_ref[...] = jnp.dot(hb, wself_ref[...], preferred_element_type=jnp.float32)


def _proj(h, W, Wself):
    bn = 1000
    return pl.pallas_call(
        _proj_body,
        grid=(N // bn,),
        in_specs=[
            pl.BlockSpec((bn, D), lambda i: (i, 0)),
            pl.BlockSpec((R, D, D), lambda i: (0, 0, 0)),
            pl.BlockSpec((D, D), lambda i: (0, 0)),
        ],
        out_specs=[
            pl.BlockSpec((bn, R, D), lambda i: (i, 0, 0)),
            pl.BlockSpec((bn, D), lambda i: (i, 0)),
        ],
        out_shape=[
            jax.ShapeDtypeStruct((N, R, D), jnp.float32),
            jax.ShapeDtypeStruct((N, D), jnp.float32),
        ],
        interpret=_INTERPRET,
    )(h, W, Wself)


def _combine_body(a0_ref, a1_ref, d0_ref, d1_ref, self_ref, out_ref):
    agg = a0_ref[...] + a1_ref[...]
    dsum = jnp.sum(d0_ref[...] + d1_ref[...], axis=1) * (1.0 / D)
    deginv = 1.0 / jnp.maximum(dsum, 1.0)
    out_ref[...] = jnp.maximum(agg * deginv[:, None] + self_ref[...], 0.0)


def _combine(a0, a1, d0, d1, selfp):
    bn = 2000
    return pl.pallas_call(
        _combine_body,
        grid=(N // bn,),
        in_specs=[
            pl.BlockSpec((bn, D), lambda i: (i, 0)),
            pl.BlockSpec((bn, D), lambda i: (i, 0)),
            pl.BlockSpec((bn, D), lambda i: (i, 0)),
            pl.BlockSpec((bn, D), lambda i: (i, 0)),
            pl.BlockSpec((bn, D), lambda i: (i, 0)),
        ],
        out_specs=pl.BlockSpec((bn, D), lambda i: (i, 0)),
        out_shape=jax.ShapeDtypeStruct((N, D), jnp.float32),
        interpret=_INTERPRET,
    )(a0, a1, d0, d1, selfp)


def _head_body(h1_ref, h2_ref, wa_ref, wb_ref, rel_ref, relemb_ref, wr_ref,
               fcb_ref, out_ref):
    i = pl.program_id(0)
    bn = h1_ref.shape[0]
    s = (jnp.dot(h1_ref[...], wa_ref[...], preferred_element_type=jnp.float32)
         + jnp.dot(h2_ref[...], wb_ref[...], preferred_element_type=jnp.float32))
    node = jax.lax.broadcasted_iota(jnp.int32, (bn, 1), 0) + i * bn
    ishead = (node % G == 0).astype(jnp.float32)
    istail = (node % G == 1).astype(jnp.float32)
    sel = jnp.concatenate(
        [jnp.full((bn, 1), 1.0 / G, jnp.float32), ishead, istail], axis=1)
    u = jnp.sum(s * sel, axis=1, keepdims=True)
    gid = (node // G)[:, 0]
    pool = (jax.lax.broadcasted_iota(jnp.int32, (B, bn), 0)
            == gid[None, :]).astype(jnp.float32)
    contrib = jnp.dot(pool, u, preferred_element_type=jnp.float32)

    @pl.when(i == 0)
    def _():
        relv = jnp.dot(relemb_ref[...], wr_ref[...],
                       preferred_element_type=jnp.float32)
        onehot = (jax.lax.broadcasted_iota(jnp.int32, (B, R), 1)
                  == rel_ref[...]).astype(jnp.float32)
        out_ref[...] = (jnp.dot(onehot, relv, preferred_element_type=jnp.float32)
                        + fcb_ref[0, 0])

    out_ref[...] += contrib


def _head(h1, h2, wa, wb, rel_labels, rel_emb, wr, fcb):
    bn = 2000
    return pl.pallas_call(
        _head_body,
        grid=(N // bn,),
        in_specs=[
            pl.BlockSpec((bn, D), lambda i: (i, 0)),
            pl.BlockSpec((bn, D), lambda i: (i, 0)),
            pl.BlockSpec((D, 3), lambda i: (0, 0)),
            pl.BlockSpec((D, 3), lambda i: (0, 0)),
            pl.BlockSpec((B, 1), lambda i: (0, 0)),
            pl.BlockSpec((R, 32), lambda i: (0, 0)),
            pl.BlockSpec((32, 1), lambda i: (0, 0)),
            pl.BlockSpec((1, 1), lambda i: (0, 0)),
        ],
        out_specs=pl.BlockSpec((B, 1), lambda i: (0, 0)),
        out_shape=jax.ShapeDtypeStruct((B, 1), jnp.float32),
        interpret=_INTERPRET,
    )(h1, h2, wa, wb, rel_labels, rel_emb, wr, fcb)


def kernel(x, edge_index, edge_type, graph_ids, head_ids, tail_ids, rel_labels,
           W1, W2, Wself1, Wself2, rel_emb, fc_W, fc_b):
    src = edge_index[0]
    dst = edge_index[1]

    pad = _EPAD - E
    src2 = jnp.concatenate([src, jnp.zeros((pad,), jnp.int32)]).reshape(-1, _CH)
    et2 = jnp.concatenate([edge_type,
                           jnp.zeros((pad,), jnp.int32)]).reshape(-1, _CH)
    dst2 = jnp.concatenate([dst,
                            jnp.full((pad,), N, jnp.int32)]).reshape(-1, _CH)

    degp = _deg_pass(dst2)
    d0, d1 = degp[0, :N], degp[1, :N]

    def layer(h, W, Wself):
        hall, selfp = _proj(h, W, Wself)
        acc = _sc_pass(hall.reshape(N * R, D), src2, et2, dst2)
        return _combine(acc[0, :N], acc[1, :N], d0, d1, selfp)

    h1 = layer(x, W1, Wself1)
    h2 = layer(h1, W2, Wself2)

    wa = jnp.stack([fc_W[0:D, 0], fc_W[2 * D:3 * D, 0],
                    fc_W[4 * D:5 * D, 0]], axis=1)
    wb = jnp.stack([fc_W[D:2 * D, 0], fc_W[3 * D:4 * D, 0],
                    fc_W[5 * D:6 * D, 0]], axis=1)
    wr = fc_W[6 * D:6 * D + 32]
    return _head(h1, h2, wa, wb, rel_labels[:, None], rel_emb, wr,
                 fc_b.reshape(1, 1))


# speedup vs baseline: 11.1081x; 1.1214x over previous
import functools

import jax
import jax.numpy as jnp
from jax.experimental import pallas as pl
from jax.experimental.pallas import tpu as pltpu
from jax.experimental.pallas import tpu_sc as plsc

N = 10000
E = 320000
D = 128
R = 8
B = 200
G = 50

_INTERPRET = False

_NW = 32
_CH = 128
_NCH0 = 112
_NCH1 = 48
_CT = 16 * (_NCH0 + _NCH1)
_EPAD = _CT * _CH
_NCHD = _CT // _NW
_ROWS = 10112
_RPS = _ROWS // 16


def _sc_body(hall, src2, et2, dst2, acc_out,
             stage_a, stage_b, gidxv, dstv, rowsv, zbuf, acc_sh):
    c = jax.lax.axis_index("c")
    s = jax.lax.axis_index("s")

    zeros16 = jnp.zeros((16,), jnp.float32)

    def zero_zbuf(i, carry):
        zbuf[i // 8, pl.ds((i % 8) * 16, 16)] = zeros16
        return carry
    jax.lax.fori_loop(0, 8 * 8, zero_zbuf, 0)

    base = s * _RPS
    def zero_acc(i, carry):
        pltpu.sync_copy(zbuf, acc_sh.at[pl.ds(base + i * 8, 8)])
        return carry
    jax.lax.fori_loop(0, _RPS // 8, zero_acc, 0)

    def run(nch, ebase):
        pltpu.sync_copy(dst2.at[pl.ds(ebase, nch)], dstv.at[pl.ds(0, nch)])

        def ggrp(g, carry):
            pltpu.sync_copy(src2.at[pl.ds(ebase + g * 8, 8)], stage_a)
            pltpu.sync_copy(et2.at[pl.ds(ebase + g * 8, 8)], stage_b)

            def gx(f, carry2):
                jj = f // 8
                sl = pl.ds((f % 8) * 16, 16)
                gidxv[g * 8 + jj, sl] = stage_a[jj, sl] * R + stage_b[jj, sl]
                return carry2
            jax.lax.fori_loop(0, 64, gx, 0)
            return carry
        jax.lax.fori_loop(0, nch // 8, ggrp, 0)

        plsc.subcore_barrier()

        def chunk(j, carry):
            pltpu.sync_copy(hall.at[gidxv.at[j]], rowsv)
            pltpu.sync_copy(rowsv, acc_sh.at[dstv.at[j]], add=True)
            return carry
        jax.lax.fori_loop(0, nch, chunk, 0)

    @pl.when(c == 0)
    def _():
        run(_NCH0, s * _NCH0)

    @pl.when(c == 1)
    def _():
        run(_NCH1, 16 * _NCH0 + s * _NCH1)

    plsc.subcore_barrier()

    pltpu.sync_copy(acc_sh.at[pl.ds(base, _RPS)],
                    acc_out.at[c, pl.ds(base, _RPS)])


def _sc_pass(hall_flat, src2, et2, dst2):
    mesh = plsc.VectorSubcoreMesh(core_axis_name="c", subcore_axis_name="s")
    f = pl.kernel(
        _sc_body,
        out_type=jax.ShapeDtypeStruct((2, _ROWS, D), jnp.float32),
        mesh=mesh,
        scratch_types=[
            pltpu.VMEM((8, _CH), jnp.int32),
            pltpu.VMEM((8, _CH), jnp.int32),
            pltpu.VMEM((_NCH0, _CH), jnp.int32),
            pltpu.VMEM((_NCH0, _CH), jnp.int32),
            pltpu.VMEM((_CH, D), jnp.float32),
            pltpu.VMEM((8, D), jnp.float32),
            pltpu.VMEM_SHARED((_ROWS, D), jnp.float32),
        ],
        interpret=_INTERPRET,
    )
    return f(hall_flat, src2, et2, dst2)


def _deg_body(dst2, deg_out, dstv, onesbuf, zbuf, deg_sh):
    c = jax.lax.axis_index("c")
    s = jax.lax.axis_index("s")
    wid = c * 16 + s

    pltpu.sync_copy(dst2.at[pl.ds(wid * _NCHD, _NCHD)], dstv)

    zeros16 = jnp.zeros((16,), jnp.float32)
    ones16 = jnp.ones((16,), jnp.float32)

    def fillz(i, carry):
        zbuf[i // 8, pl.ds((i % 8) * 16, 16)] = zeros16
        return carry
    jax.lax.fori_loop(0, 8 * 8, fillz, 0)

    def fillo(i, carry):
        onesbuf[i // 8, pl.ds((i % 8) * 16, 16)] = ones16
        return carry
    jax.lax.fori_loop(0, _CH * 8, fillo, 0)

    base = s * _RPS
    def zero_deg(i, carry):
        pltpu.sync_copy(zbuf, deg_sh.at[pl.ds(base + i * 8, 8)])
        return carry
    jax.lax.fori_loop(0, _RPS // 8, zero_deg, 0)

    plsc.subcore_barrier()

    def chunk(j, carry):
        pltpu.sync_copy(onesbuf, deg_sh.at[dstv.at[j]], add=True)
        return carry
    jax.lax.fori_loop(0, _NCHD, chunk, 0)

    plsc.subcore_barrier()

    pltpu.sync_copy(deg_sh.at[pl.ds(base, _RPS)],
                    deg_out.at[c, pl.ds(base, _RPS)])


def _deg_pass(dst2):
    mesh = plsc.VectorSubcoreMesh(core_axis_name="c", subcore_axis_name="s")
    f = pl.kernel(
        _deg_body,
        out_type=jax.ShapeDtypeStruct((2, _ROWS, D), jnp.float32),
        mesh=mesh,
        scratch_types=[
            pltpu.VMEM((_NCHD, _CH), jnp.int32),
            pltpu.VMEM((_CH, D), jnp.float32),
            pltpu.VMEM((8, D), jnp.float32),
            pltpu.VMEM_SHARED((_ROWS, D), jnp.float32),
        ],
        interpret=_INTERPRET,
    )
    return f(dst2)


def _proj_body(h_ref, w_ref, wself_ref, hall_ref, self_ref):
    hb = h_ref[...]
    for r in range(R):
        hall_ref[:, r, :] = jnp.dot(hb, w_ref[r],
                                    preferred_element_type=jnp.float32)
    self_ref[...] = jnp.dot(hb, wself_ref[...], preferred_element_type=jnp.float32)


def _proj(h, W, Wself):
    bn = 1000
    return pl.pallas_call(
        _proj_body,
        grid=(N // bn,),
        in_specs=[
            pl.BlockSpec((bn, D), lambda i: (i, 0)),
            pl.BlockSpec((R, D, D), lambda i: (0, 0, 0)),
            pl.BlockSpec((D, D), lambda i: (0, 0)),
        ],
        out_specs=[
            pl.BlockSpec((bn, R, D), lambda i: (i, 0, 0)),
            pl.BlockSpec((bn, D), lambda i: (i, 0)),
        ],
        out_shape=[
            jax.ShapeDtypeStruct((N, R, D), jnp.float32),
            jax.ShapeDtypeStruct((N, D), jnp.float32),
        ],
        interpret=_INTERPRET,
    )(h, W, Wself)


def _combine_body(a0_ref, a1_ref, d0_ref, d1_ref, self_ref, out_ref):
    agg = a0_ref[...] + a1_ref[...]
    dsum = jnp.sum(d0_ref[...] + d1_ref[...], axis=1) * (1.0 / D)
    deginv = 1.0 / jnp.maximum(dsum, 1.0)
    out_ref[...] = jnp.maximum(agg * deginv[:, None] + self_ref[...], 0.0)


def _combine(a0, a1, d0, d1, selfp):
    bn = 2000
    return pl.pallas_call(
        _combine_body,
        grid=(N // bn,),
        in_specs=[
            pl.BlockSpec((bn, D), lambda i: (i, 0)),
            pl.BlockSpec((bn, D), lambda i: (i, 0)),
            pl.BlockSpec((bn, D), lambda i: (i, 0)),
            pl.BlockSpec((bn, D), lambda i: (i, 0)),
            pl.BlockSpec((bn, D), lambda i: (i, 0)),
        ],
        out_specs=pl.BlockSpec((bn, D), lambda i: (i, 0)),
        out_shape=jax.ShapeDtypeStruct((N, D), jnp.float32),
        interpret=_INTERPRET,
    )(a0, a1, d0, d1, selfp)


def _head_body(h1_ref, h2_ref, wa_ref, wb_ref, rel_ref, relemb_ref, wr_ref,
               fcb_ref, out_ref):
    i = pl.program_id(0)
    bn = h1_ref.shape[0]
    s = (jnp.dot(h1_ref[...], wa_ref[...], preferred_element_type=jnp.float32)
         + jnp.dot(h2_ref[...], wb_ref[...], preferred_element_type=jnp.float32))
    node = jax.lax.broadcasted_iota(jnp.int32, (bn, 1), 0) + i * bn
    ishead = (node % G == 0).astype(jnp.float32)
    istail = (node % G == 1).astype(jnp.float32)
    sel = jnp.concatenate(
        [jnp.full((bn, 1), 1.0 / G, jnp.float32), ishead, istail], axis=1)
    u = jnp.sum(s * sel, axis=1, keepdims=True)
    gid = (node // G)[:, 0]
    pool = (jax.lax.broadcasted_iota(jnp.int32, (B, bn), 0)
            == gid[None, :]).astype(jnp.float32)
    contrib = jnp.dot(pool, u, preferred_element_type=jnp.float32)

    @pl.when(i == 0)
    def _():
        relv = jnp.dot(relemb_ref[...], wr_ref[...],
                       preferred_element_type=jnp.float32)
        onehot = (jax.lax.broadcasted_iota(jnp.int32, (B, R), 1)
                  == rel_ref[...]).astype(jnp.float32)
        out_ref[...] = (jnp.dot(onehot, relv, preferred_element_type=jnp.float32)
                        + fcb_ref[0, 0])

    out_ref[...] += contrib


def _head(h1, h2, wa, wb, rel_labels, rel_emb, wr, fcb):
    bn = 2000
    return pl.pallas_call(
        _head_body,
        grid=(N // bn,),
        in_specs=[
            pl.BlockSpec((bn, D), lambda i: (i, 0)),
            pl.BlockSpec((bn, D), lambda i: (i, 0)),
            pl.BlockSpec((D, 3), lambda i: (0, 0)),
            pl.BlockSpec((D, 3), lambda i: (0, 0)),
            pl.BlockSpec((B, 1), lambda i: (0, 0)),
            pl.BlockSpec((R, 32), lambda i: (0, 0)),
            pl.BlockSpec((32, 1), lambda i: (0, 0)),
            pl.BlockSpec((1, 1), lambda i: (0, 0)),
        ],
        out_specs=pl.BlockSpec((B, 1), lambda i: (0, 0)),
        out_shape=jax.ShapeDtypeStruct((B, 1), jnp.float32),
        interpret=_INTERPRET,
    )(h1, h2, wa, wb, rel_labels, rel_emb, wr, fcb)


def kernel(x, edge_index, edge_type, graph_ids, head_ids, tail_ids, rel_labels,
           W1, W2, Wself1, Wself2, rel_emb, fc_W, fc_b):
    src = edge_index[0]
    dst = edge_index[1]

    pad = _EPAD - E
    src2 = jnp.concatenate([src, jnp.zeros((pad,), jnp.int32)]).reshape(-1, _CH)
    et2 = jnp.concatenate([edge_type,
                           jnp.zeros((pad,), jnp.int32)]).reshape(-1, _CH)
    dst2 = jnp.concatenate([dst,
                            jnp.full((pad,), N, jnp.int32)]).reshape(-1, _CH)

    degp = _deg_pass(dst2)
    d0, d1 = degp[0, :N], degp[1, :N]

    def layer(h, W, Wself):
        hall, selfp = _proj(h, W, Wself)
        acc = _sc_pass(hall.reshape(N * R, D), src2, et2, dst2)
        return _combine(acc[0, :N], acc[1, :N], d0, d1, selfp)

    h1 = layer(x, W1, Wself1)
    h2 = layer(h1, W2, Wself2)

    wa = jnp.stack([fc_W[0:D, 0], fc_W[2 * D:3 * D, 0],
                    fc_W[4 * D:5 * D, 0]], axis=1)
    wb = jnp.stack([fc_W[D:2 * D, 0], fc_W[3 * D:4 * D, 0],
                    fc_W[5 * D:6 * D, 0]], axis=1)
    wr = fc_W[6 * D:6 * D + 32]
    return _head(h1, h2, wa, wb, rel_labels[:, None], rel_emb, wr,
                 fc_b.reshape(1, 1))


# speedup vs baseline: 11.4388x; 1.0298x over previous
import functools

import jax
import jax.numpy as jnp
from jax.experimental import pallas as pl
from jax.experimental.pallas import tpu as pltpu
from jax.experimental.pallas import tpu_sc as plsc

N = 10000
E = 320000
D = 128
R = 8
B = 200
G = 50

_INTERPRET = False

_NW = 32
_CH = 128
_NCH0 = 112
_NCH1 = 48
_CT = 16 * (_NCH0 + _NCH1)
_EPAD = _CT * _CH
_NCHD = _CT // _NW
_ROWS = 10112
_RPS = _ROWS // 16


def _sc_body(hall, src2, et2, dst2, acc_out,
             stage_a, stage_b, gidx8, dst8, rows_a, rows_b, zbuf,
             gsem_a, gsem_b, ssem_a, ssem_b, acc_sh):
    c = jax.lax.axis_index("c")
    s = jax.lax.axis_index("s")

    zeros16 = jnp.zeros((16,), jnp.float32)

    def zero_zbuf(i, carry):
        zbuf[i // 8, pl.ds((i % 8) * 16, 16)] = zeros16
        return carry
    jax.lax.fori_loop(0, 8 * 8, zero_zbuf, 0)

    base = s * _RPS
    def zero_acc(i, carry):
        pltpu.sync_copy(zbuf, acc_sh.at[pl.ds(base + i * 8, 8)])
        return carry
    jax.lax.fori_loop(0, _RPS // 8, zero_acc, 0)

    plsc.subcore_barrier()

    rows = (rows_a, rows_b)
    gsem = (gsem_a, gsem_b)
    ssem = (ssem_a, ssem_b)

    def run(nch, ebase):
        def group(g, carry):
            @pl.when(g > 0)
            def _():
                pltpu.make_async_copy(rows_a, acc_sh.at[dst8.at[0]],
                                      ssem_a).wait()
                pltpu.make_async_copy(rows_b, acc_sh.at[dst8.at[0]],
                                      ssem_b).wait()

            gb = ebase + g * 8
            pltpu.sync_copy(src2.at[pl.ds(gb, 8)], stage_a)
            pltpu.sync_copy(et2.at[pl.ds(gb, 8)], stage_b)
            pltpu.sync_copy(dst2.at[pl.ds(gb, 8)], dst8)

            def gx(f, carry2):
                jj = f // 8
                sl = pl.ds((f % 8) * 16, 16)
                gidx8[jj, sl] = stage_a[jj, sl] * R + stage_b[jj, sl]
                return carry2
            jax.lax.fori_loop(0, 64, gx, 0)

            pltpu.async_copy(hall.at[gidx8.at[0]], rows_a, gsem_a)
            for jj in range(8):
                p, q = jj % 2, (jj + 1) % 2
                pltpu.make_async_copy(hall.at[gidx8.at[jj]], rows[p],
                                      gsem[p]).wait()
                if jj < 7:
                    if jj >= 1:
                        pltpu.make_async_copy(rows[q], acc_sh.at[dst8.at[0]],
                                              ssem[q]).wait()
                    pltpu.async_copy(hall.at[gidx8.at[jj + 1]], rows[q],
                                     gsem[q])
                pltpu.async_copy(rows[p], acc_sh.at[dst8.at[jj]], ssem[p],
                                 add=True)
            return carry
        jax.lax.fori_loop(0, nch // 8, group, 0)

        pltpu.make_async_copy(rows_a, acc_sh.at[dst8.at[0]], ssem_a).wait()
        pltpu.make_async_copy(rows_b, acc_sh.at[dst8.at[0]], ssem_b).wait()

    @pl.when(c == 0)
    def _():
        run(_NCH0, s * _NCH0)

    @pl.when(c == 1)
    def _():
        run(_NCH1, 16 * _NCH0 + s * _NCH1)

    plsc.subcore_barrier()

    pltpu.sync_copy(acc_sh.at[pl.ds(base, _RPS)],
                    acc_out.at[c, pl.ds(base, _RPS)])


def _sc_pass(hall_flat, src2, et2, dst2):
    mesh = plsc.VectorSubcoreMesh(core_axis_name="c", subcore_axis_name="s")
    f = pl.kernel(
        _sc_body,
        out_type=jax.ShapeDtypeStruct((2, _ROWS, D), jnp.float32),
        mesh=mesh,
        scratch_types=[
            pltpu.VMEM((8, _CH), jnp.int32),
            pltpu.VMEM((8, _CH), jnp.int32),
            pltpu.VMEM((8, _CH), jnp.int32),
            pltpu.VMEM((8, _CH), jnp.int32),
            pltpu.VMEM((_CH, D), jnp.float32),
            pltpu.VMEM((_CH, D), jnp.float32),
            pltpu.VMEM((8, D), jnp.float32),
            pltpu.SemaphoreType.DMA,
            pltpu.SemaphoreType.DMA,
            pltpu.SemaphoreType.DMA,
            pltpu.SemaphoreType.DMA,
            pltpu.VMEM_SHARED((_ROWS, D), jnp.float32),
        ],
        interpret=_INTERPRET,
    )
    return f(hall_flat, src2, et2, dst2)


def _deg_body(dst2, deg_out, dstv, onesbuf, zbuf, deg_sh):
    c = jax.lax.axis_index("c")
    s = jax.lax.axis_index("s")
    wid = c * 16 + s

    pltpu.sync_copy(dst2.at[pl.ds(wid * _NCHD, _NCHD)], dstv)

    zeros16 = jnp.zeros((16,), jnp.float32)
    ones16 = jnp.ones((16,), jnp.float32)

    def fillz(i, carry):
        zbuf[i // 8, pl.ds((i % 8) * 16, 16)] = zeros16
        return carry
    jax.lax.fori_loop(0, 8 * 8, fillz, 0)

    def fillo(i, carry):
        onesbuf[i // 8, pl.ds((i % 8) * 16, 16)] = ones16
        return carry
    jax.lax.fori_loop(0, _CH * 8, fillo, 0)

    base = s * _RPS
    def zero_deg(i, carry):
        pltpu.sync_copy(zbuf, deg_sh.at[pl.ds(base + i * 8, 8)])
        return carry
    jax.lax.fori_loop(0, _RPS // 8, zero_deg, 0)

    plsc.subcore_barrier()

    def chunk(j, carry):
        pltpu.sync_copy(onesbuf, deg_sh.at[dstv.at[j]], add=True)
        return carry
    jax.lax.fori_loop(0, _NCHD, chunk, 0)

    plsc.subcore_barrier()

    pltpu.sync_copy(deg_sh.at[pl.ds(base, _RPS)],
                    deg_out.at[c, pl.ds(base, _RPS)])


def _deg_pass(dst2):
    mesh = plsc.VectorSubcoreMesh(core_axis_name="c", subcore_axis_name="s")
    f = pl.kernel(
        _deg_body,
        out_type=jax.ShapeDtypeStruct((2, _ROWS, D), jnp.float32),
        mesh=mesh,
        scratch_types=[
            pltpu.VMEM((_NCHD, _CH), jnp.int32),
            pltpu.VMEM((_CH, D), jnp.float32),
            pltpu.VMEM((8, D), jnp.float32),
            pltpu.VMEM_SHARED((_ROWS, D), jnp.float32),
        ],
        interpret=_INTERPRET,
    )
    return f(dst2)


def _proj_body(h_ref, w_ref, wself_ref, hall_ref, self_ref):
    hb = h_ref[...]
    for r in range(R):
        hall_ref[:, r, :] = jnp.dot(hb, w_ref[r],
                                    preferred_element_type=jnp.float32)
    self_ref[...] = jnp.dot(hb, wself_ref[...], preferred_element_type=jnp.float32)


def _proj(h, W, Wself):
    bn = 1000
    return pl.pallas_call(
        _proj_body,
        grid=(N // bn,),
        in_specs=[
            pl.BlockSpec((bn, D), lambda i: (i, 0)),
            pl.BlockSpec((R, D, D), lambda i: (0, 0, 0)),
            pl.BlockSpec((D, D), lambda i: (0, 0)),
        ],
        out_specs=[
            pl.BlockSpec((bn, R, D), lambda i: (i, 0, 0)),
            pl.BlockSpec((bn, D), lambda i: (i, 0)),
        ],
        out_shape=[
            jax.ShapeDtypeStruct((N, R, D), jnp.float32),
            jax.ShapeDtypeStruct((N, D), jnp.float32),
        ],
        interpret=_INTERPRET,
    )(h, W, Wself)


def _combine_body(a0_ref, a1_ref, d0_ref, d1_ref, self_ref, out_ref):
    agg = a0_ref[...] + a1_ref[...]
    dsum = jnp.sum(d0_ref[...] + d1_ref[...], axis=1) * (1.0 / D)
    deginv = 1.0 / jnp.maximum(dsum, 1.0)
    out_ref[...] = jnp.maximum(agg * deginv[:, None] + self_ref[...], 0.0)


def _combine(a0, a1, d0, d1, selfp):
    bn = 2000
    return pl.pallas_call(
        _combine_body,
        grid=(N // bn,),
        in_specs=[
            pl.BlockSpec((bn, D), lambda i: (i, 0)),
            pl.BlockSpec((bn, D), lambda i: (i, 0)),
            pl.BlockSpec((bn, D), lambda i: (i, 0)),
            pl.BlockSpec((bn, D), lambda i: (i, 0)),
            pl.BlockSpec((bn, D), lambda i: (i, 0)),
        ],
        out_specs=pl.BlockSpec((bn, D), lambda i: (i, 0)),
        out_shape=jax.ShapeDtypeStruct((N, D), jnp.float32),
        interpret=_INTERPRET,
    )(a0, a1, d0, d1, selfp)


def _head_body(h1_ref, h2_ref, wa_ref, wb_ref, rel_ref, relemb_ref, wr_ref,
               fcb_ref, out_ref):
    i = pl.program_id(0)
    bn = h1_ref.shape[0]
    s = (jnp.dot(h1_ref[...], wa_ref[...], preferred_element_type=jnp.float32)
         + jnp.dot(h2_ref[...], wb_ref[...], preferred_element_type=jnp.float32))
    node = jax.lax.broadcasted_iota(jnp.int32, (bn, 1), 0) + i * bn
    ishead = (node % G == 0).astype(jnp.float32)
    istail = (node % G == 1).astype(jnp.float32)
    sel = jnp.concatenate(
        [jnp.full((bn, 1), 1.0 / G, jnp.float32), ishead, istail], axis=1)
    u = jnp.sum(s * sel, axis=1, keepdims=True)
    gid = (node // G)[:, 0]
    pool = (jax.lax.broadcasted_iota(jnp.int32, (B, bn), 0)
            == gid[None, :]).astype(jnp.float32)
    contrib = jnp.dot(pool, u, preferred_element_type=jnp.float32)

    @pl.when(i == 0)
    def _():
        relv = jnp.dot(relemb_ref[...], wr_ref[...],
                       preferred_element_type=jnp.float32)
        onehot = (jax.lax.broadcasted_iota(jnp.int32, (B, R), 1)
                  == rel_ref[...]).astype(jnp.float32)
        out_ref[...] = (jnp.dot(onehot, relv, preferred_element_type=jnp.float32)
                        + fcb_ref[0, 0])

    out_ref[...] += contrib


def _head(h1, h2, wa, wb, rel_labels, rel_emb, wr, fcb):
    bn = 2000
    return pl.pallas_call(
        _head_body,
        grid=(N // bn,),
        in_specs=[
            pl.BlockSpec((bn, D), lambda i: (i, 0)),
            pl.BlockSpec((bn, D), lambda i: (i, 0)),
            pl.BlockSpec((D, 3), lambda i: (0, 0)),
            pl.BlockSpec((D, 3), lambda i: (0, 0)),
            pl.BlockSpec((B, 1), lambda i: (0, 0)),
            pl.BlockSpec((R, 32), lambda i: (0, 0)),
            pl.BlockSpec((32, 1), lambda i: (0, 0)),
            pl.BlockSpec((1, 1), lambda i: (0, 0)),
        ],
        out_specs=pl.BlockSpec((B, 1), lambda i: (0, 0)),
        out_shape=jax.ShapeDtypeStruct((B, 1), jnp.float32),
        interpret=_INTERPRET,
    )(h1, h2, wa, wb, rel_labels, rel_emb, wr, fcb)


def kernel(x, edge_index, edge_type, graph_ids, head_ids, tail_ids, rel_labels,
           W1, W2, Wself1, Wself2, rel_emb, fc_W, fc_b):
    src = edge_index[0]
    dst = edge_index[1]

    pad = _EPAD - E
    src2 = jnp.concatenate([src, jnp.zeros((pad,), jnp.int32)]).reshape(-1, _CH)
    et2 = jnp.concatenate([edge_type,
                           jnp.zeros((pad,), jnp.int32)]).reshape(-1, _CH)
    dst2 = jnp.concatenate([dst,
                            jnp.full((pad,), N, jnp.int32)]).reshape(-1, _CH)

    degp = _deg_pass(dst2)
    d0, d1 = degp[0, :N], degp[1, :N]

    def layer(h, W, Wself):
        hall, selfp = _proj(h, W, Wself)
        acc = _sc_pass(hall.reshape(N * R, D), src2, et2, dst2)
        return _combine(acc[0, :N], acc[1, :N], d0, d1, selfp)

    h1 = layer(x, W1, Wself1)
    h2 = layer(h1, W2, Wself2)

    wa = jnp.stack([fc_W[0:D, 0], fc_W[2 * D:3 * D, 0],
                    fc_W[4 * D:5 * D, 0]], axis=1)
    wb = jnp.stack([fc_W[D:2 * D, 0], fc_W[3 * D:4 * D, 0],
                    fc_W[5 * D:6 * D, 0]], axis=1)
    wr = fc_W[6 * D:6 * D + 32]
    return _head(h1, h2, wa, wb, rel_labels[:, None], rel_emb, wr,
                 fc_b.reshape(1, 1))


# speedup vs baseline: 11.6211x; 1.0159x over previous
import functools

import jax
import jax.numpy as jnp
from jax.experimental import pallas as pl
from jax.experimental.pallas import tpu as pltpu
from jax.experimental.pallas import tpu_sc as plsc

N = 10000
E = 320000
D = 128
R = 8
B = 200
G = 50

_INTERPRET = False

_NW = 32
_CH = 128
_NCH0 = 128
_NCH1 = 32
_CT = 16 * (_NCH0 + _NCH1)
_EPAD = _CT * _CH
_NCHD = _CT // _NW
_ROWS = 10112
_RPS = _ROWS // 16


def _sc_body(hall, src2, et2, dst2, acc_out,
             stage_a, stage_b, gidx8, dst8, rows_a, rows_b, zbuf,
             gsem_a, gsem_b, ssem_a, ssem_b, acc_sh):
    c = jax.lax.axis_index("c")
    s = jax.lax.axis_index("s")

    zeros16 = jnp.zeros((16,), jnp.float32)

    def zero_zbuf(i, carry):
        zbuf[i // 8, pl.ds((i % 8) * 16, 16)] = zeros16
        return carry
    jax.lax.fori_loop(0, 8 * 8, zero_zbuf, 0)

    base = s * _RPS
    def zero_acc(i, carry):
        pltpu.sync_copy(zbuf, acc_sh.at[pl.ds(base + i * 8, 8)])
        return carry
    jax.lax.fori_loop(0, _RPS // 8, zero_acc, 0)

    plsc.subcore_barrier()

    rows = (rows_a, rows_b)
    gsem = (gsem_a, gsem_b)
    ssem = (ssem_a, ssem_b)

    def run(nch, ebase):
        def group(g, carry):
            @pl.when(g > 0)
            def _():
                pltpu.make_async_copy(rows_a, acc_sh.at[dst8.at[0]],
                                      ssem_a).wait()
                pltpu.make_async_copy(rows_b, acc_sh.at[dst8.at[0]],
                                      ssem_b).wait()

            gb = ebase + g * 8
            pltpu.sync_copy(src2.at[pl.ds(gb, 8)], stage_a)
            pltpu.sync_copy(et2.at[pl.ds(gb, 8)], stage_b)
            pltpu.sync_copy(dst2.at[pl.ds(gb, 8)], dst8)

            def gx(f, carry2):
                jj = f // 8
                sl = pl.ds((f % 8) * 16, 16)
                gidx8[jj, sl] = stage_a[jj, sl] * R + stage_b[jj, sl]
                return carry2
            jax.lax.fori_loop(0, 64, gx, 0)

            pltpu.async_copy(hall.at[gidx8.at[0]], rows_a, gsem_a)
            for jj in range(8):
                p, q = jj % 2, (jj + 1) % 2
                pltpu.make_async_copy(hall.at[gidx8.at[jj]], rows[p],
                                      gsem[p]).wait()
                if jj < 7:
                    if jj >= 1:
                        pltpu.make_async_copy(rows[q], acc_sh.at[dst8.at[0]],
                                              ssem[q]).wait()
                    pltpu.async_copy(hall.at[gidx8.at[jj + 1]], rows[q],
                                     gsem[q])
                pltpu.async_copy(rows[p], acc_sh.at[dst8.at[jj]], ssem[p],
                                 add=True)
            return carry
        jax.lax.fori_loop(0, nch // 8, group, 0)

        pltpu.make_async_copy(rows_a, acc_sh.at[dst8.at[0]], ssem_a).wait()
        pltpu.make_async_copy(rows_b, acc_sh.at[dst8.at[0]], ssem_b).wait()

    @pl.when(c == 0)
    def _():
        run(_NCH0, s * _NCH0)

    @pl.when(c == 1)
    def _():
        run(_NCH1, 16 * _NCH0 + s * _NCH1)

    plsc.subcore_barrier()

    pltpu.sync_copy(acc_sh.at[pl.ds(base, _RPS)],
                    acc_out.at[c, pl.ds(base, _RPS)])


def _sc_pass(hall_flat, src2, et2, dst2):
    mesh = plsc.VectorSubcoreMesh(core_axis_name="c", subcore_axis_name="s")
    f = pl.kernel(
        _sc_body,
        out_type=jax.ShapeDtypeStruct((2, _ROWS, D), jnp.float32),
        mesh=mesh,
        scratch_types=[
            pltpu.VMEM((8, _CH), jnp.int32),
            pltpu.VMEM((8, _CH), jnp.int32),
            pltpu.VMEM((8, _CH), jnp.int32),
            pltpu.VMEM((8, _CH), jnp.int32),
            pltpu.VMEM((_CH, D), jnp.float32),
            pltpu.VMEM((_CH, D), jnp.float32),
            pltpu.VMEM((8, D), jnp.float32),
            pltpu.SemaphoreType.DMA,
            pltpu.SemaphoreType.DMA,
            pltpu.SemaphoreType.DMA,
            pltpu.SemaphoreType.DMA,
            pltpu.VMEM_SHARED((_ROWS, D), jnp.float32),
        ],
        interpret=_INTERPRET,
    )
    return f(hall_flat, src2, et2, dst2)


def _deg_body(dst2, deg_out, dstv, onesbuf, zbuf, deg_sh):
    c = jax.lax.axis_index("c")
    s = jax.lax.axis_index("s")
    wid = c * 16 + s

    pltpu.sync_copy(dst2.at[pl.ds(wid * _NCHD, _NCHD)], dstv)

    zeros16 = jnp.zeros((16,), jnp.float32)
    ones16 = jnp.ones((16,), jnp.float32)

    def fillz(i, carry):
        zbuf[i // 8, pl.ds((i % 8) * 16, 16)] = zeros16
        return carry
    jax.lax.fori_loop(0, 8 * 8, fillz, 0)

    def fillo(i, carry):
        onesbuf[i // 8, pl.ds((i % 8) * 16, 16)] = ones16
        return carry
    jax.lax.fori_loop(0, _CH * 8, fillo, 0)

    base = s * _RPS
    def zero_deg(i, carry):
        pltpu.sync_copy(zbuf, deg_sh.at[pl.ds(base + i * 8, 8)])
        return carry
    jax.lax.fori_loop(0, _RPS // 8, zero_deg, 0)

    plsc.subcore_barrier()

    def chunk(j, carry):
        pltpu.sync_copy(onesbuf, deg_sh.at[dstv.at[j]], add=True)
        return carry
    jax.lax.fori_loop(0, _NCHD, chunk, 0)

    plsc.subcore_barrier()

    pltpu.sync_copy(deg_sh.at[pl.ds(base, _RPS)],
                    deg_out.at[c, pl.ds(base, _RPS)])


def _deg_pass(dst2):
    mesh = plsc.VectorSubcoreMesh(core_axis_name="c", subcore_axis_name="s")
    f = pl.kernel(
        _deg_body,
        out_type=jax.ShapeDtypeStruct((2, _ROWS, D), jnp.float32),
        mesh=mesh,
        scratch_types=[
            pltpu.VMEM((_NCHD, _CH), jnp.int32),
            pltpu.VMEM((_CH, D), jnp.float32),
            pltpu.VMEM((8, D), jnp.float32),
            pltpu.VMEM_SHARED((_ROWS, D), jnp.float32),
        ],
        interpret=_INTERPRET,
    )
    return f(dst2)


def _proj_body(h_ref, w_ref, wself_ref, hall_ref, self_ref):
    hb = h_ref[...]
    for r in range(R):
        hall_ref[:, r, :] = jnp.dot(hb, w_ref[r],
                                    preferred_element_type=jnp.float32)
    self_ref[...] = jnp.dot(hb, wself_ref[...], preferred_element_type=jnp.float32)


def _proj(h, W, Wself):
    bn = 1000
    return pl.pallas_call(
        _proj_body,
        grid=(N // bn,),
        in_specs=[
            pl.BlockSpec((bn, D), lambda i: (i, 0)),
            pl.BlockSpec((R, D, D), lambda i: (0, 0, 0)),
            pl.BlockSpec((D, D), lambda i: (0, 0)),
        ],
        out_specs=[
            pl.BlockSpec((bn, R, D), lambda i: (i, 0, 0)),
            pl.BlockSpec((bn, D), lambda i: (i, 0)),
        ],
        out_shape=[
            jax.ShapeDtypeStruct((N, R, D), jnp.float32),
            jax.ShapeDtypeStruct((N, D), jnp.float32),
        ],
        interpret=_INTERPRET,
    )(h, W, Wself)


def _combine_body(acc_ref, deg_ref, self_ref, out_ref):
    agg = acc_ref[0] + acc_ref[1]
    dsum = jnp.sum(deg_ref[0] + deg_ref[1], axis=1) * (1.0 / D)
    deginv = 1.0 / jnp.maximum(dsum, 1.0)
    out_ref[...] = jnp.maximum(agg * deginv[:, None] + self_ref[...], 0.0)


def _combine(acc, degp, selfp):
    bn = 2000
    return pl.pallas_call(
        _combine_body,
        grid=(N // bn,),
        in_specs=[
            pl.BlockSpec((2, bn, D), lambda i: (0, i, 0)),
            pl.BlockSpec((2, bn, D), lambda i: (0, i, 0)),
            pl.BlockSpec((bn, D), lambda i: (i, 0)),
        ],
        out_specs=pl.BlockSpec((bn, D), lambda i: (i, 0)),
        out_shape=jax.ShapeDtypeStruct((N, D), jnp.float32),
        interpret=_INTERPRET,
    )(acc, degp, selfp)


def _head_body(h1_ref, h2_ref, wa_ref, wb_ref, rel_ref, relemb_ref, wr_ref,
               fcb_ref, out_ref):
    i = pl.program_id(0)
    bn = h1_ref.shape[0]
    s = (jnp.dot(h1_ref[...], wa_ref[...], preferred_element_type=jnp.float32)
         + jnp.dot(h2_ref[...], wb_ref[...], preferred_element_type=jnp.float32))
    node = jax.lax.broadcasted_iota(jnp.int32, (bn, 1), 0) + i * bn
    ishead = (node % G == 0).astype(jnp.float32)
    istail = (node % G == 1).astype(jnp.float32)
    sel = jnp.concatenate(
        [jnp.full((bn, 1), 1.0 / G, jnp.float32), ishead, istail], axis=1)
    u = jnp.sum(s * sel, axis=1, keepdims=True)
    gid = (node // G)[:, 0]
    pool = (jax.lax.broadcasted_iota(jnp.int32, (B, bn), 0)
            == gid[None, :]).astype(jnp.float32)
    contrib = jnp.dot(pool, u, preferred_element_type=jnp.float32)

    @pl.when(i == 0)
    def _():
        relv = jnp.dot(relemb_ref[...], wr_ref[...],
                       preferred_element_type=jnp.float32)
        onehot = (jax.lax.broadcasted_iota(jnp.int32, (B, R), 1)
                  == rel_ref[...]).astype(jnp.float32)
        out_ref[...] = (jnp.dot(onehot, relv, preferred_element_type=jnp.float32)
                        + fcb_ref[0, 0])

    out_ref[...] += contrib


def _head(h1, h2, wa, wb, rel_labels, rel_emb, wr, fcb):
    bn = 2000
    return pl.pallas_call(
        _head_body,
        grid=(N // bn,),
        in_specs=[
            pl.BlockSpec((bn, D), lambda i: (i, 0)),
            pl.BlockSpec((bn, D), lambda i: (i, 0)),
            pl.BlockSpec((D, 3), lambda i: (0, 0)),
            pl.BlockSpec((D, 3), lambda i: (0, 0)),
            pl.BlockSpec((B, 1), lambda i: (0, 0)),
            pl.BlockSpec((R, 32), lambda i: (0, 0)),
            pl.BlockSpec((32, 1), lambda i: (0, 0)),
            pl.BlockSpec((1, 1), lambda i: (0, 0)),
        ],
        out_specs=pl.BlockSpec((B, 1), lambda i: (0, 0)),
        out_shape=jax.ShapeDtypeStruct((B, 1), jnp.float32),
        interpret=_INTERPRET,
    )(h1, h2, wa, wb, rel_labels, rel_emb, wr, fcb)


def kernel(x, edge_index, edge_type, graph_ids, head_ids, tail_ids, rel_labels,
           W1, W2, Wself1, Wself2, rel_emb, fc_W, fc_b):
    src = edge_index[0]
    dst = edge_index[1]

    pad = _EPAD - E
    src2 = jnp.concatenate([src, jnp.zeros((pad,), jnp.int32)]).reshape(-1, _CH)
    et2 = jnp.concatenate([edge_type,
                           jnp.zeros((pad,), jnp.int32)]).reshape(-1, _CH)
    dst2 = jnp.concatenate([dst,
                            jnp.full((pad,), N, jnp.int32)]).reshape(-1, _CH)

    degp = _deg_pass(dst2)

    def layer(h, W, Wself):
        hall, selfp = _proj(h, W, Wself)
        acc = _sc_pass(hall.reshape(N * R, D), src2, et2, dst2)
        return _combine(acc, degp, selfp)

    h1 = layer(x, W1, Wself1)
    h2 = layer(h1, W2, Wself2)

    wa = jnp.stack([fc_W[0:D, 0], fc_W[2 * D:3 * D, 0],
                    fc_W[4 * D:5 * D, 0]], axis=1)
    wb = jnp.stack([fc_W[D:2 * D, 0], fc_W[3 * D:4 * D, 0],
                    fc_W[5 * D:6 * D, 0]], axis=1)
    wr = fc_W[6 * D:6 * D + 32]
    return _head(h1, h2, wa, wb, rel_labels[:, None], rel_emb, wr,
                 fc_b.reshape(1, 1))
